# baseline (speedup 1.0000x reference)
import jax
import jax.numpy as jnp
from jax.experimental import pallas as pl
from jax.experimental.pallas import tpu as pltpu

_N = 1024
_K = 1024
_D = 64
_NB = 256


def _body(x_ref, c_ref, o_ref, ct_ref):
    @pl.when(pl.program_id(0) == 0)
    def _():
        ct_ref[:] = c_ref[:].T

    xb = x_ref[:]
    ct = ct_ref[:]
    acc = jnp.zeros((_NB, _K), jnp.float32)
    for d in range(_D):
        acc = acc + jnp.abs(xb[:, d][:, None] - ct[d, :][None, :])
    logits = acc * acc * (-0.5)
    m = jnp.max(logits, axis=1, keepdims=True)
    e = jnp.exp(logits - m)
    s = jnp.sum(e, axis=1, keepdims=True)
    o_ref[:] = e / s


def kernel(x, placeCells):
    x = jnp.reshape(x, (-1, _D))
    return pl.pallas_call(
        _body,
        grid=(_N // _NB,),
        in_specs=[
            pl.BlockSpec((_NB, _D), lambda i: (i, 0)),
            pl.BlockSpec((_K, _D), lambda i: (0, 0)),
        ],
        out_specs=pl.BlockSpec((_NB, _K), lambda i: (i, 0)),
        out_shape=jax.ShapeDtypeStruct((_N, _K), jnp.float32),
        scratch_shapes=[pltpu.VMEM((_D, _K), jnp.float32)],
    )(x, placeCells)


# speedup vs baseline: 1.0661x; 1.0661x over previous
import jax
import jax.numpy as jnp
from jax.experimental import pallas as pl

_N = 1024
_K = 1024
_D = 64
_NB = 256


def _body(x_ref, ct_ref, o_ref):
    xb = x_ref[:]
    ct = ct_ref[:]
    acc = jnp.zeros((_NB, _K), jnp.float32)
    for d in range(_D):
        acc = acc + jnp.abs(xb[:, d][:, None] - ct[d, :][None, :])
    logits = acc * acc * (-0.5)
    m = jnp.max(logits, axis=1, keepdims=True)
    e = jnp.exp(logits - m)
    s = jnp.sum(e, axis=1, keepdims=True)
    o_ref[:] = e / s


def kernel(x, placeCells):
    x = jnp.reshape(x, (-1, _D))
    ct = placeCells.T
    return pl.pallas_call(
        _body,
        grid=(_N // _NB,),
        in_specs=[
            pl.BlockSpec((_NB, _D), lambda i: (i, 0)),
            pl.BlockSpec((_D, _K), lambda i: (0, 0)),
        ],
        out_specs=pl.BlockSpec((_NB, _K), lambda i: (i, 0)),
        out_shape=jax.ShapeDtypeStruct((_N, _K), jnp.float32),
    )(x, ct)
